# baseline (speedup 1.0000x reference)
import jax
import jax.numpy as jnp
from jax import lax
from jax.experimental import pallas as pl
from jax.experimental.pallas import tpu as pltpu
from jax.experimental.pallas import tpu_sc as plsc

_N_CLS = 5
_K = 10
_Q = 10
_EPB = 16
_CH = 128


def _make_sc_segsum(n_rows, D, n_cores, n_subcores):
    rows_per_ep = _N_CLS * _K
    n_ep = n_rows // rows_per_ep
    ep_per_core = n_ep // n_cores
    ep_per_tile = ep_per_core // n_subcores
    rows_per_tile = ep_per_tile * rows_per_ep
    segs_per_core = ep_per_core * _N_CLS
    segs_per_tile = ep_per_tile * _N_CLS
    n_chunks = rows_per_tile // _CH

    mesh = plsc.VectorSubcoreMesh(core_axis_name="c", subcore_axis_name="s",
                                  num_cores=n_cores, num_subcores=n_subcores)

    def body(sup_hbm, lab_hbm, out_hbm, rows_v, lab_v, idx_v, zrow_v, acc_sh):
        c = lax.axis_index("c")
        s = lax.axis_index("s")

        zero16 = jnp.zeros((16,), jnp.float32)
        for i in range(16):
            for j in range(D // 16):
                zrow_v[i, pl.ds(j * 16, 16)] = zero16
        for k in range(segs_per_tile // 16):
            pltpu.sync_copy(zrow_v, acc_sh.at[pl.ds(s * segs_per_tile + k * 16, 16)])
        plsc.subcore_barrier()

        tile_row0 = (c * ep_per_core + s * ep_per_tile) * rows_per_ep
        seg_base = c * segs_per_core

        for ci in range(n_chunks):
            base = tile_row0 + ci * _CH
            pltpu.sync_copy(sup_hbm.at[pl.ds(base, _CH)], rows_v)
            pltpu.sync_copy(lab_hbm.at[pl.ds(base, _CH)], lab_v)
            for j in range(_CH // 16):
                rid = lax.iota(jnp.int32, 16) + (base + j * 16)
                ep = lax.div(rid, jnp.int32(rows_per_ep))
                lab16 = lab_v[pl.ds(j * 16, 16)]
                idx_v[pl.ds(j * 16, 16)] = (ep * _N_CLS - seg_base) + lab16
            pltpu.sync_copy(rows_v, acc_sh.at[idx_v], add=True)

        plsc.subcore_barrier()
        out0 = c * segs_per_core + s * segs_per_tile
        pltpu.sync_copy(acc_sh.at[pl.ds(s * segs_per_tile, segs_per_tile)],
                        out_hbm.at[pl.ds(out0, segs_per_tile)])

    return pl.kernel(
        body,
        out_type=jax.ShapeDtypeStruct((n_ep * _N_CLS, D), jnp.float32),
        mesh=mesh,
        scratch_types=[
            pltpu.VMEM((_CH, D), jnp.float32),
            pltpu.VMEM((_CH,), jnp.int32),
            pltpu.VMEM((_CH,), jnp.int32),
            pltpu.VMEM((16, D), jnp.float32),
            pltpu.VMEM_SHARED((segs_per_core, D), jnp.float32),
        ],
    )


def _sim_block(sums_ref, q_ref, lab_ref, w_ref, out_ref):
    E = _EPB
    R = E * _N_CLS * _K
    S = E * _N_CLS
    W = w_ref[:]

    lab = lab_ref[0]
    ep = lax.broadcasted_iota(jnp.int32, (1, R), 1) // (_N_CLS * _K)
    seg = ep * _N_CLS + lab
    seg_ids = lax.broadcasted_iota(jnp.int32, (S, 1), 0)
    onehot_t = (seg_ids == seg).astype(jnp.float32)
    counts = jnp.sum(onehot_t, axis=1, keepdims=True)

    mean_s = sums_ref[:] / jnp.maximum(counts, 1.0)
    proto = jnp.dot(mean_s, W, preferred_element_type=jnp.float32)

    emb_q = jnp.dot(q_ref[:], W, preferred_element_type=jnp.float32)

    qn = jnp.sqrt(jnp.sum(emb_q * emb_q, axis=1, keepdims=True))
    pn2 = lax.dot_general(
        jnp.ones((1, W.shape[0]), jnp.float32), proto * proto,
        (((1,), (1,)), ((), ())), preferred_element_type=jnp.float32)
    pn = jnp.sqrt(pn2)

    num = lax.dot_general(
        emb_q, proto, (((1,), (1,)), ((), ())),
        preferred_element_type=jnp.float32)
    sims = num / (qn * pn + 1e-8)

    row_ep = lax.broadcasted_iota(jnp.int32, (R, S), 0) // (_N_CLS * _Q)
    col_ep = lax.broadcasted_iota(jnp.int32, (R, S), 1) // _N_CLS
    masked = sims * (row_ep == col_ep).astype(jnp.float32)
    sel = (lax.broadcasted_iota(jnp.int32, (S, _N_CLS), 0) % _N_CLS
           == lax.broadcasted_iota(jnp.int32, (S, _N_CLS), 1)
           ).astype(jnp.float32)
    out_ref[:] = jnp.dot(masked, sel, preferred_element_type=jnp.float32)


def kernel(supports, queries, support_labels, W):
    n_rows, D = supports.shape
    rows_per_ep = _N_CLS * _K
    B = n_rows // rows_per_ep
    E = _EPB
    n_blocks = B // E
    R = E * rows_per_ep
    S = E * _N_CLS

    info = plsc.get_sparse_core_info()
    sc_segsum = _make_sc_segsum(n_rows, D, info.num_cores, info.num_subcores)
    sums = sc_segsum(supports, support_labels)

    labels3 = support_labels.reshape(n_blocks, 1, R)

    out = pl.pallas_call(
        _sim_block,
        grid=(n_blocks,),
        in_specs=[
            pl.BlockSpec((S, D), lambda i: (i, 0)),
            pl.BlockSpec((R, D), lambda i: (i, 0)),
            pl.BlockSpec((1, 1, R), lambda i: (i, 0, 0)),
            pl.BlockSpec((D, D), lambda i: (0, 0)),
        ],
        out_specs=pl.BlockSpec((R, _N_CLS), lambda i: (i, 0)),
        out_shape=jax.ShapeDtypeStruct((n_rows, _N_CLS), jnp.float32),
    )(sums, queries, labels3, W)

    return out.reshape(-1)


# speedup vs baseline: 1.1579x; 1.1579x over previous
import jax
import jax.numpy as jnp
from jax import lax
from jax.experimental import pallas as pl
from jax.experimental.pallas import tpu as pltpu
from jax.experimental.pallas import tpu_sc as plsc

_N_CLS = 5
_K = 10
_Q = 10
_EPB = 16
_CH = 128


def _make_sc_segsum(n_rows, D, n_cores, n_subcores):
    rows_per_ep = _N_CLS * _K
    n_ep = n_rows // rows_per_ep
    ep_per_core = n_ep // n_cores
    ep_per_tile = ep_per_core // n_subcores
    rows_per_tile = ep_per_tile * rows_per_ep
    segs_per_core = ep_per_core * _N_CLS
    segs_per_tile = ep_per_tile * _N_CLS
    n_chunks = rows_per_tile // _CH

    mesh = plsc.VectorSubcoreMesh(core_axis_name="c", subcore_axis_name="s",
                                  num_cores=n_cores, num_subcores=n_subcores)

    NBUF = 3

    def body(sup_hbm, lab_hbm, out_hbm, rows_v, lab_v, idx_v, zrow_v, acc_sh,
             gsems, ssems):
        c = lax.axis_index("c")
        s = lax.axis_index("s")

        zero16 = jnp.zeros((16,), jnp.float32)
        for i in range(16):
            for j in range(D // 16):
                zrow_v[i, pl.ds(j * 16, 16)] = zero16
        for k in range(segs_per_tile // 16):
            pltpu.sync_copy(zrow_v, acc_sh.at[pl.ds(s * segs_per_tile + k * 16, 16)])
        plsc.subcore_barrier()

        tile_row0 = (c * ep_per_core + s * ep_per_tile) * rows_per_ep
        seg_base = c * segs_per_core

        gathers = [None] * n_chunks
        scatters = [None] * n_chunks

        def start_gather(ci):
            b = ci % NBUF
            base = tile_row0 + ci * _CH
            g1 = pltpu.async_copy(sup_hbm.at[pl.ds(base, _CH)],
                                  rows_v.at[b], gsems.at[b])
            g2 = pltpu.async_copy(lab_hbm.at[pl.ds(base, _CH)],
                                  lab_v.at[b], gsems.at[b])
            gathers[ci] = (g1, g2)

        for ci in range(NBUF):
            start_gather(ci)

        for ci in range(n_chunks):
            b = ci % NBUF
            base = tile_row0 + ci * _CH
            if ci >= 1 and ci + NBUF - 1 < n_chunks:
                scatters[ci - 1].wait()
                start_gather(ci + NBUF - 1)
            for g in gathers[ci]:
                g.wait()
            for j in range(_CH // 16):
                rid = lax.iota(jnp.int32, 16) + (base + j * 16)
                ep = lax.div(rid, jnp.int32(rows_per_ep))
                lab16 = lab_v[b, pl.ds(j * 16, 16)]
                idx_v[b, pl.ds(j * 16, 16)] = (ep * _N_CLS - seg_base) + lab16
            scatters[ci] = pltpu.async_copy(
                rows_v.at[b], acc_sh.at[idx_v.at[b]], ssems.at[b], add=True)

        for ci in range(n_chunks - NBUF, n_chunks):
            if scatters[ci] is not None:
                scatters[ci].wait()

        plsc.subcore_barrier()
        out0 = c * segs_per_core + s * segs_per_tile
        pltpu.sync_copy(acc_sh.at[pl.ds(s * segs_per_tile, segs_per_tile)],
                        out_hbm.at[pl.ds(out0, segs_per_tile)])

    return pl.kernel(
        body,
        out_type=jax.ShapeDtypeStruct((n_ep * _N_CLS, D), jnp.float32),
        mesh=mesh,
        scratch_types=[
            pltpu.VMEM((NBUF, _CH, D), jnp.float32),
            pltpu.VMEM((NBUF, _CH), jnp.int32),
            pltpu.VMEM((NBUF, _CH), jnp.int32),
            pltpu.VMEM((16, D), jnp.float32),
            pltpu.VMEM_SHARED((segs_per_core, D), jnp.float32),
            pltpu.SemaphoreType.DMA((NBUF,)),
            pltpu.SemaphoreType.DMA((NBUF,)),
        ],
    )


def _sim_block(sums_ref, q_ref, lab_ref, w_ref, out_ref):
    E = _EPB
    R = E * _N_CLS * _K
    S = E * _N_CLS
    W = w_ref[:]

    lab = lab_ref[0]
    ep = lax.broadcasted_iota(jnp.int32, (1, R), 1) // (_N_CLS * _K)
    seg = ep * _N_CLS + lab
    seg_ids = lax.broadcasted_iota(jnp.int32, (S, 1), 0)
    onehot_t = (seg_ids == seg).astype(jnp.float32)
    counts = jnp.sum(onehot_t, axis=1, keepdims=True)

    mean_s = sums_ref[:] / jnp.maximum(counts, 1.0)
    proto = jnp.dot(mean_s, W, preferred_element_type=jnp.float32)

    emb_q = jnp.dot(q_ref[:], W, preferred_element_type=jnp.float32)

    qn = jnp.sqrt(jnp.sum(emb_q * emb_q, axis=1, keepdims=True))
    pn2 = lax.dot_general(
        jnp.ones((1, W.shape[0]), jnp.float32), proto * proto,
        (((1,), (1,)), ((), ())), preferred_element_type=jnp.float32)
    pn = jnp.sqrt(pn2)

    num = lax.dot_general(
        emb_q, proto, (((1,), (1,)), ((), ())),
        preferred_element_type=jnp.float32)
    sims = num / (qn * pn + 1e-8)

    row_ep = lax.broadcasted_iota(jnp.int32, (R, S), 0) // (_N_CLS * _Q)
    col_ep = lax.broadcasted_iota(jnp.int32, (R, S), 1) // _N_CLS
    masked = sims * (row_ep == col_ep).astype(jnp.float32)
    sel = (lax.broadcasted_iota(jnp.int32, (S, _N_CLS), 0) % _N_CLS
           == lax.broadcasted_iota(jnp.int32, (S, _N_CLS), 1)
           ).astype(jnp.float32)
    out_ref[:] = jnp.dot(masked, sel, preferred_element_type=jnp.float32)


def kernel(supports, queries, support_labels, W):
    n_rows, D = supports.shape
    rows_per_ep = _N_CLS * _K
    B = n_rows // rows_per_ep
    E = _EPB
    n_blocks = B // E
    R = E * rows_per_ep
    S = E * _N_CLS

    info = plsc.get_sparse_core_info()
    sc_segsum = _make_sc_segsum(n_rows, D, info.num_cores, info.num_subcores)
    sums = sc_segsum(supports, support_labels)

    labels3 = support_labels.reshape(n_blocks, 1, R)

    out = pl.pallas_call(
        _sim_block,
        grid=(n_blocks,),
        in_specs=[
            pl.BlockSpec((S, D), lambda i: (i, 0)),
            pl.BlockSpec((R, D), lambda i: (i, 0)),
            pl.BlockSpec((1, 1, R), lambda i: (i, 0, 0)),
            pl.BlockSpec((D, D), lambda i: (0, 0)),
        ],
        out_specs=pl.BlockSpec((R, _N_CLS), lambda i: (i, 0)),
        out_shape=jax.ShapeDtypeStruct((n_rows, _N_CLS), jnp.float32),
    )(sums, queries, labels3, W)

    return out.reshape(-1)


# speedup vs baseline: 1.1747x; 1.0145x over previous
import jax
import jax.numpy as jnp
from jax import lax
from jax.experimental import pallas as pl
from jax.experimental.pallas import tpu as pltpu
from jax.experimental.pallas import tpu_sc as plsc

_N_CLS = 5
_K = 10
_Q = 10
_EPB = 16
_CH = 128


def _make_sc_segsum(n_rows, D, n_cores, n_subcores):
    rows_per_ep = _N_CLS * _K
    n_ep = n_rows // rows_per_ep
    ep_per_core = n_ep // n_cores
    ep_per_tile = ep_per_core // n_subcores
    rows_per_tile = ep_per_tile * rows_per_ep
    segs_per_core = ep_per_core * _N_CLS
    segs_per_tile = ep_per_tile * _N_CLS
    n_chunks = rows_per_tile // _CH

    mesh = plsc.VectorSubcoreMesh(core_axis_name="c", subcore_axis_name="s",
                                  num_cores=n_cores, num_subcores=n_subcores)

    NBUF = 3

    def body(sup_hbm, lab_hbm, out_hbm, rows_v, lab_v, idx_v, zrow_v, acc_sh,
             gsems, ssems):
        c = lax.axis_index("c")
        s = lax.axis_index("s")

        zero16 = jnp.zeros((16,), jnp.float32)
        for i in range(16):
            for j in range(D // 16):
                zrow_v[i, pl.ds(j * 16, 16)] = zero16
        for k in range(segs_per_tile // 16):
            pltpu.sync_copy(zrow_v, acc_sh.at[pl.ds(s * segs_per_tile + k * 16, 16)])
        plsc.subcore_barrier()

        tile_row0 = (c * ep_per_core + s * ep_per_tile) * rows_per_ep
        seg_base = c * segs_per_core

        gathers = [None] * n_chunks
        scatters = [None] * n_chunks

        def start_gather(ci):
            b = ci % NBUF
            base = tile_row0 + ci * _CH
            g1 = pltpu.async_copy(sup_hbm.at[pl.ds(base, _CH)],
                                  rows_v.at[b], gsems.at[b])
            g2 = pltpu.async_copy(lab_hbm.at[pl.ds(base, _CH)],
                                  lab_v.at[b], gsems.at[b])
            gathers[ci] = (g1, g2)

        for ci in range(NBUF):
            start_gather(ci)

        for ci in range(n_chunks):
            b = ci % NBUF
            base = tile_row0 + ci * _CH
            if ci >= 1 and ci + NBUF - 1 < n_chunks:
                scatters[ci - 1].wait()
                start_gather(ci + NBUF - 1)
            for g in gathers[ci]:
                g.wait()
            for j in range(_CH // 16):
                rid = lax.iota(jnp.int32, 16) + (base + j * 16)
                ep = lax.div(rid, jnp.int32(rows_per_ep))
                lab16 = lab_v[b, pl.ds(j * 16, 16)]
                idx_v[b, pl.ds(j * 16, 16)] = (ep * _N_CLS - seg_base) + lab16
            scatters[ci] = pltpu.async_copy(
                rows_v.at[b], acc_sh.at[idx_v.at[b]], ssems.at[b], add=True)

        for ci in range(n_chunks - NBUF, n_chunks):
            if scatters[ci] is not None:
                scatters[ci].wait()

        plsc.subcore_barrier()
        out0 = c * segs_per_core + s * segs_per_tile
        pltpu.sync_copy(acc_sh.at[pl.ds(s * segs_per_tile, segs_per_tile)],
                        out_hbm.at[pl.ds(out0, segs_per_tile)])

    return pl.kernel(
        body,
        out_type=jax.ShapeDtypeStruct((n_ep * _N_CLS, D), jnp.float32),
        mesh=mesh,
        scratch_types=[
            pltpu.VMEM((NBUF, _CH, D), jnp.float32),
            pltpu.VMEM((NBUF, _CH), jnp.int32),
            pltpu.VMEM((NBUF, _CH), jnp.int32),
            pltpu.VMEM((16, D), jnp.float32),
            pltpu.VMEM_SHARED((segs_per_core, D), jnp.float32),
            pltpu.SemaphoreType.DMA((NBUF,)),
            pltpu.SemaphoreType.DMA((NBUF,)),
        ],
    )


def _sim_block(sums_ref, q_ref, lab_ref, w_ref, out_ref):
    E = _EPB
    R = E * _N_CLS * _K
    S = E * _N_CLS
    W = w_ref[:]

    lab = lab_ref[0]
    ep = lax.broadcasted_iota(jnp.int32, (1, R), 1) // (_N_CLS * _K)
    seg = ep * _N_CLS + lab
    seg_ids = lax.broadcasted_iota(jnp.int32, (S, 1), 0)
    onehot_t = (seg_ids == seg).astype(jnp.float32)
    counts = jnp.sum(onehot_t, axis=1, keepdims=True)

    mean_s = sums_ref[:] / jnp.maximum(counts, 1.0)
    Wb = W.astype(jnp.bfloat16)
    proto = jnp.dot(mean_s.astype(jnp.bfloat16), Wb,
                    preferred_element_type=jnp.float32)

    emb_q = jnp.dot(q_ref[:].astype(jnp.bfloat16), Wb,
                    preferred_element_type=jnp.float32)

    qn = jnp.sqrt(jnp.sum(emb_q * emb_q, axis=1, keepdims=True))
    pn2 = lax.dot_general(
        jnp.ones((1, W.shape[0]), jnp.float32), proto * proto,
        (((1,), (1,)), ((), ())), preferred_element_type=jnp.float32)
    pn = jnp.sqrt(pn2)

    num = lax.dot_general(
        emb_q, proto, (((1,), (1,)), ((), ())),
        preferred_element_type=jnp.float32)
    sims = num / (qn * pn + 1e-8)

    row_ep = lax.broadcasted_iota(jnp.int32, (R, S), 0) // (_N_CLS * _Q)
    col_ep = lax.broadcasted_iota(jnp.int32, (R, S), 1) // _N_CLS
    masked = sims * (row_ep == col_ep).astype(jnp.float32)
    sel = (lax.broadcasted_iota(jnp.int32, (S, _N_CLS), 0) % _N_CLS
           == lax.broadcasted_iota(jnp.int32, (S, _N_CLS), 1)
           ).astype(jnp.float32)
    out_ref[:] = jnp.dot(masked, sel, preferred_element_type=jnp.float32)


def kernel(supports, queries, support_labels, W):
    n_rows, D = supports.shape
    rows_per_ep = _N_CLS * _K
    B = n_rows // rows_per_ep
    E = _EPB
    n_blocks = B // E
    R = E * rows_per_ep
    S = E * _N_CLS

    info = plsc.get_sparse_core_info()
    sc_segsum = _make_sc_segsum(n_rows, D, info.num_cores, info.num_subcores)
    sums = sc_segsum(supports, support_labels)

    labels3 = support_labels.reshape(n_blocks, 1, R)

    out = pl.pallas_call(
        _sim_block,
        grid=(n_blocks,),
        in_specs=[
            pl.BlockSpec((S, D), lambda i: (i, 0)),
            pl.BlockSpec((R, D), lambda i: (i, 0)),
            pl.BlockSpec((1, 1, R), lambda i: (i, 0, 0)),
            pl.BlockSpec((D, D), lambda i: (0, 0)),
        ],
        out_specs=pl.BlockSpec((R, _N_CLS), lambda i: (i, 0)),
        out_shape=jax.ShapeDtypeStruct((n_rows, _N_CLS), jnp.float32),
    )(sums, queries, labels3, W)

    return out.reshape(-1)
